# Initial kernel scaffold; baseline (speedup 1.0000x reference)
import jax
import jax.numpy as jnp
from jax.experimental import pallas as pl


def kernel(input, adj, degree, W, b):
    raise NotImplementedError("write your pallas kernel here")


# speedup vs baseline: 6.2240x; 6.2240x over previous
import jax
import jax.numpy as jnp
from jax.experimental import pallas as pl

_BM = 256


def _support_body(x_ref, wt_ref, b_ref, out_ref):
    out_ref[...] = (
        jnp.dot(x_ref[...], wt_ref[...], preferred_element_type=jnp.float32)
        + b_ref[...]
    )


def _agg_body(adj_ref, sup1_ref, sup2_ref, d1_ref, d2_ref, top_ref, bot_ref):
    i = pl.program_id(0)
    a = adj_ref[...]
    top_ref[...] = d1_ref[...] * jnp.dot(
        a, sup2_ref[...], preferred_element_type=jnp.float32
    )
    contrib = jnp.dot(a.T, sup1_ref[...], preferred_element_type=jnp.float32)

    @pl.when(i == 0)
    def _init():
        bot_ref[...] = contrib

    @pl.when(i > 0)
    def _acc():
        bot_ref[...] += contrib

    @pl.when(i == pl.num_programs(0) - 1)
    def _scale():
        bot_ref[...] *= d2_ref[...]


def kernel(input, adj, degree, W, b):
    n1, n2 = adj.shape
    h = W.shape[0]

    sup = pl.pallas_call(
        _support_body,
        out_shape=jax.ShapeDtypeStruct((n1 + n2, h), jnp.float32),
        in_specs=[
            pl.BlockSpec((n1 + n2, W.shape[1]), lambda: (0, 0)),
            pl.BlockSpec((W.shape[1], h), lambda: (0, 0)),
            pl.BlockSpec((1, h), lambda: (0, 0)),
        ],
        out_specs=pl.BlockSpec((n1 + n2, h), lambda: (0, 0)),
    )(input, W.T, b.reshape(1, h))

    sup1 = sup[:n1]
    sup2 = sup[n1:]
    d1 = degree[:n1].reshape(n1, 1)
    d2 = degree[n1:].reshape(n2, 1)

    grid = n1 // _BM
    top, bot = pl.pallas_call(
        _agg_body,
        grid=(grid,),
        out_shape=(
            jax.ShapeDtypeStruct((n1, h), jnp.float32),
            jax.ShapeDtypeStruct((n2, h), jnp.float32),
        ),
        in_specs=[
            pl.BlockSpec((_BM, n2), lambda i: (i, 0)),
            pl.BlockSpec((_BM, h), lambda i: (i, 0)),
            pl.BlockSpec((n2, h), lambda i: (0, 0)),
            pl.BlockSpec((_BM, 1), lambda i: (i, 0)),
            pl.BlockSpec((n2, 1), lambda i: (0, 0)),
        ],
        out_specs=(
            pl.BlockSpec((_BM, h), lambda i: (i, 0)),
            pl.BlockSpec((n2, h), lambda i: (0, 0)),
        ),
    )(adj, sup1, sup2, d1, d2)

    return jnp.concatenate([top, bot], axis=0)


# speedup vs baseline: 7.0720x; 1.1362x over previous
import jax
import jax.numpy as jnp
from jax.experimental import pallas as pl
from jax.experimental.pallas import tpu as pltpu

_BM = 512


def _support_body(x_ref, wt_ref, b_ref, sup_ref, sup1t_ref):
    n1 = sup1t_ref.shape[1]
    sup = (
        jnp.dot(x_ref[...], wt_ref[...], preferred_element_type=jnp.float32)
        + b_ref[...]
    )
    sup_ref[...] = sup
    sup1t_ref[...] = sup[:n1].T


def _agg_body(adj_ref, sup1t_ref, sup2_ref, d1_ref, d2_ref, top_ref, bot_ref,
              acc_ref):
    i = pl.program_id(0)
    a = adj_ref[...]
    top_ref[...] = d1_ref[...] * jnp.dot(
        a, sup2_ref[...], preferred_element_type=jnp.float32
    )
    contrib = jnp.dot(
        sup1t_ref[...], a, preferred_element_type=jnp.float32
    )

    @pl.when(i == 0)
    def _init():
        acc_ref[...] = contrib

    @pl.when(i > 0)
    def _acc():
        acc_ref[...] += contrib

    @pl.when(i == pl.num_programs(0) - 1)
    def _finish():
        bot_ref[...] = d2_ref[...] * acc_ref[...].T


def kernel(input, adj, degree, W, b):
    n1, n2 = adj.shape
    h = W.shape[0]

    sup, sup1t = pl.pallas_call(
        _support_body,
        out_shape=(
            jax.ShapeDtypeStruct((n1 + n2, h), jnp.float32),
            jax.ShapeDtypeStruct((h, n1), jnp.float32),
        ),
        in_specs=[
            pl.BlockSpec((n1 + n2, W.shape[1]), lambda: (0, 0)),
            pl.BlockSpec((W.shape[1], h), lambda: (0, 0)),
            pl.BlockSpec((1, h), lambda: (0, 0)),
        ],
        out_specs=(
            pl.BlockSpec((n1 + n2, h), lambda: (0, 0)),
            pl.BlockSpec((h, n1), lambda: (0, 0)),
        ),
    )(input, W.T, b.reshape(1, h))

    sup2 = sup[n1:]
    d1 = degree[:n1].reshape(n1, 1)
    d2 = degree[n1:].reshape(n2, 1)

    grid = n1 // _BM
    top, bot = pl.pallas_call(
        _agg_body,
        grid=(grid,),
        out_shape=(
            jax.ShapeDtypeStruct((n1, h), jnp.float32),
            jax.ShapeDtypeStruct((n2, h), jnp.float32),
        ),
        in_specs=[
            pl.BlockSpec((_BM, n2), lambda i: (i, 0)),
            pl.BlockSpec((h, _BM), lambda i: (0, i)),
            pl.BlockSpec((n2, h), lambda i: (0, 0)),
            pl.BlockSpec((_BM, 1), lambda i: (i, 0)),
            pl.BlockSpec((n2, 1), lambda i: (0, 0)),
        ],
        out_specs=(
            pl.BlockSpec((_BM, h), lambda i: (i, 0)),
            pl.BlockSpec((n2, h), lambda i: (0, 0)),
        ),
        scratch_shapes=[pltpu.VMEM((h, n2), jnp.float32)],
    )(adj, sup1t, sup2, d1, d2)

    return jnp.concatenate([top, bot], axis=0)


# speedup vs baseline: 8.0381x; 1.1366x over previous
import jax
import jax.numpy as jnp
from jax.experimental import pallas as pl
from jax.experimental.pallas import tpu as pltpu

_BM = 512


def _support_body(x_ref, wt_ref, b_ref, sup2_ref, sup1t_ref):
    n1 = sup1t_ref.shape[1]
    sup = (
        jnp.dot(x_ref[...], wt_ref[...], preferred_element_type=jnp.float32)
        + b_ref[...]
    )
    sup2_ref[...] = sup[n1:].astype(jnp.bfloat16)
    sup1t_ref[...] = sup[:n1].T.astype(jnp.bfloat16)


def _agg_body(adj_ref, sup1t_ref, sup2_ref, d1_ref, d2_ref, top_ref, bot_ref,
              acc_ref):
    i = pl.program_id(0)
    a = adj_ref[...].astype(jnp.bfloat16)
    top_ref[...] = d1_ref[...] * jnp.dot(
        a, sup2_ref[...], preferred_element_type=jnp.float32
    )
    contrib = jnp.dot(
        sup1t_ref[...], a, preferred_element_type=jnp.float32
    )

    @pl.when(i == 0)
    def _init():
        acc_ref[...] = contrib

    @pl.when(i > 0)
    def _acc():
        acc_ref[...] += contrib

    @pl.when(i == pl.num_programs(0) - 1)
    def _finish():
        bot_ref[...] = d2_ref[...] * acc_ref[...].T


def kernel(input, adj, degree, W, b):
    n1, n2 = adj.shape
    h = W.shape[0]

    sup2, sup1t = pl.pallas_call(
        _support_body,
        out_shape=(
            jax.ShapeDtypeStruct((n2, h), jnp.bfloat16),
            jax.ShapeDtypeStruct((h, n1), jnp.bfloat16),
        ),
        in_specs=[
            pl.BlockSpec((n1 + n2, W.shape[1]), lambda: (0, 0)),
            pl.BlockSpec((W.shape[1], h), lambda: (0, 0)),
            pl.BlockSpec((1, h), lambda: (0, 0)),
        ],
        out_specs=(
            pl.BlockSpec((n2, h), lambda: (0, 0)),
            pl.BlockSpec((h, n1), lambda: (0, 0)),
        ),
    )(input, W.T, b.reshape(1, h))

    d1 = degree[:n1].reshape(n1, 1)
    d2 = degree[n1:].reshape(n2, 1)

    grid = n1 // _BM
    top, bot = pl.pallas_call(
        _agg_body,
        grid=(grid,),
        out_shape=(
            jax.ShapeDtypeStruct((n1, h), jnp.float32),
            jax.ShapeDtypeStruct((n2, h), jnp.float32),
        ),
        in_specs=[
            pl.BlockSpec((_BM, n2), lambda i: (i, 0)),
            pl.BlockSpec((h, _BM), lambda i: (0, i)),
            pl.BlockSpec((n2, h), lambda i: (0, 0)),
            pl.BlockSpec((_BM, 1), lambda i: (i, 0)),
            pl.BlockSpec((n2, 1), lambda i: (0, 0)),
        ],
        out_specs=(
            pl.BlockSpec((_BM, h), lambda i: (i, 0)),
            pl.BlockSpec((n2, h), lambda i: (0, 0)),
        ),
        scratch_shapes=[pltpu.VMEM((h, n2), jnp.float32)],
    )(adj, sup1t, sup2, d1, d2)

    return jnp.concatenate([top, bot], axis=0)
